# Initial kernel scaffold; baseline (speedup 1.0000x reference)
import jax
import jax.numpy as jnp
from jax import lax
from jax.experimental import pallas as pl
from jax.experimental.pallas import tpu as pltpu
from jax.experimental.pallas import tpu_sc as plsc

_N = 10000
_E = 320000
_NFEAT = 128
_NHID = 16
_NHEAD = 8
_NCLASS = 64

_NC = 2
_NS = 16
_NW = _NC * _NS
_L = 16
_CH = 128
_NP = 10240
_AH = 8
_BN = 256


def _sc_edge_pass(H, HO):
    n_chunks = _E // _CH
    O = HO // H
    nz = _NP // _CH // _NS

    mesh = plsc.VectorSubcoreMesh(core_axis_name="c", subcore_axis_name="s",
                                  num_cores=_NC, num_subcores=_NS)

    def body(adj, hmat, asrc, adst, num_out, den_out,
             sidx, didx, hrows, asr, adr, wem, num_s, den_s, sem):
        c = lax.axis_index("c")
        s = lax.axis_index("s")
        wid = s * _NC + c
        zero16 = jnp.zeros((_L,), jnp.float32)
        iota = lax.iota(jnp.int32, _L)

        def zrow(k, carry):
            for j in range(HO // _L):
                hrows[k, pl.ds(j * _L, _L)] = zero16
            return carry
        lax.fori_loop(0, _CH, zrow, 0)
        r_pat = lax.shift_right_logical(iota, 3)
        c_pat = lax.bitwise_and(iota, 7)
        for i in range(_CH // 2):
            plsc.store_scatter(wem, [r_pat + 2 * i, c_pat], zero16)
        for j in range(nz):
            r = (s * nz + j) * _CH
            pltpu.sync_copy(hrows, num_s.at[pl.ds(r, _CH)])
            pltpu.sync_copy(wem, den_s.at[pl.ds(r, _CH)])
        plsc.subcore_barrier()

        cols = [jnp.full((_L,), h, jnp.int32) for h in range(H)]

        def chunk(i, carry):
            base = (wid + _NW * i) * _CH
            pltpu.sync_copy(adj.at[0, pl.ds(base, _CH)], sidx)
            pltpu.sync_copy(adj.at[1, pl.ds(base, _CH)], didx)
            cp1 = pltpu.async_copy(hmat.at[sidx], hrows, sem)
            cp2 = pltpu.async_copy(asrc.at[sidx], asr, sem)
            cp3 = pltpu.async_copy(adst.at[didx], adr, sem)
            cp1.wait()
            cp2.wait()
            cp3.wait()
            for g in range(_CH // _L):
                ridx = iota + (g * _L)
                for h in range(H):
                    ev = (plsc.load_gather(asr, [ridx, cols[h]])
                          + plsc.load_gather(adr, [ridx, cols[h]]))
                    ev = jnp.maximum(ev, 0.2 * ev)
                    plsc.store_scatter(wem, [ridx, cols[h]], jnp.exp(ev))
            def mul_body(k, carry2):
                for h in range(H):
                    w_s = wem[k, h]
                    for o in range(O // _L):
                        sl = pl.ds(h * O + o * _L, _L)
                        hrows[k, sl] = hrows[k, sl] * w_s
                return carry2
            lax.fori_loop(0, _CH, mul_body, 0)
            pltpu.sync_copy(hrows, num_s.at[didx], add=True)
            pltpu.sync_copy(wem, den_s.at[didx], add=True)
            return carry

        n_my = (n_chunks - wid + _NW - 1) // _NW
        lax.fori_loop(0, n_my, chunk, 0)

        plsc.subcore_barrier()
        for j in range(nz):
            r = (s * nz + j) * _CH
            pltpu.sync_copy(num_s.at[pl.ds(r, _CH)], hrows)
            pltpu.sync_copy(hrows, num_out.at[c, pl.ds(r, _CH)])
            pltpu.sync_copy(den_s.at[pl.ds(r, _CH)], wem)
            pltpu.sync_copy(wem, den_out.at[c, pl.ds(r, _CH)])

    return pl.kernel(
        body,
        out_type=(jax.ShapeDtypeStruct((_NC, _NP, HO), jnp.float32),
                  jax.ShapeDtypeStruct((_NC, _NP, _AH), jnp.float32)),
        mesh=mesh,
        scratch_types=[
            pltpu.VMEM((_CH,), jnp.int32),
            pltpu.VMEM((_CH,), jnp.int32),
            pltpu.VMEM((_CH, HO), jnp.float32),
            pltpu.VMEM((_CH, _AH), jnp.float32),
            pltpu.VMEM((_CH, _AH), jnp.float32),
            pltpu.VMEM((_CH, _AH), jnp.float32),
            pltpu.VMEM_SHARED((_NP, HO), jnp.float32),
            pltpu.VMEM_SHARED((_NP, _AH), jnp.float32),
            pltpu.SemaphoreType.DMA,
        ],
    )


def _tc_layer1(x, W1f, A1s, A1d):
    grid = ((_N + _BN - 1) // _BN,)

    def body(x_ref, w_ref, as_ref, ad_ref, h_ref, oas_ref, oad_ref):
        h = jnp.dot(x_ref[...], w_ref[...], preferred_element_type=jnp.float32)
        h_ref[...] = h
        oas_ref[...] = jnp.dot(h, as_ref[...], preferred_element_type=jnp.float32)
        oad_ref[...] = jnp.dot(h, ad_ref[...], preferred_element_type=jnp.float32)

    return pl.pallas_call(
        body,
        grid=grid,
        in_specs=[pl.BlockSpec((_BN, _NFEAT), lambda i: (i, 0)),
                  pl.BlockSpec((_NFEAT, _NFEAT), lambda i: (0, 0)),
                  pl.BlockSpec((_NFEAT, _AH), lambda i: (0, 0)),
                  pl.BlockSpec((_NFEAT, _AH), lambda i: (0, 0))],
        out_specs=[pl.BlockSpec((_BN, _NFEAT), lambda i: (i, 0)),
                   pl.BlockSpec((_BN, _AH), lambda i: (i, 0)),
                   pl.BlockSpec((_BN, _AH), lambda i: (i, 0))],
        out_shape=[jax.ShapeDtypeStruct((_N, _NFEAT), jnp.float32),
                   jax.ShapeDtypeStruct((_N, _AH), jnp.float32),
                   jax.ShapeDtypeStruct((_N, _AH), jnp.float32)],
    )(x, W1f, A1s, A1d)


def _tc_mid(num1, den1, b1r, EXPAND, W2f, A2s, A2d):
    grid = (_NP // _BN,)

    def body(n_ref, d_ref, b_ref, e_ref, w_ref, as_ref, ad_ref,
             h2_ref, oas_ref, oad_ref):
        num = n_ref[0] + n_ref[1]
        den = d_ref[0] + d_ref[1]
        den_w = jnp.dot(den, e_ref[...], preferred_element_type=jnp.float32)
        out1 = num / (den_w + 1e-16) + b_ref[...]
        h1 = jnp.where(out1 > 0, out1, jnp.expm1(out1))
        h2 = jnp.dot(h1, w_ref[...], preferred_element_type=jnp.float32)
        h2_ref[...] = h2
        oas_ref[...] = jnp.dot(h2, as_ref[...], preferred_element_type=jnp.float32)
        oad_ref[...] = jnp.dot(h2, ad_ref[...], preferred_element_type=jnp.float32)

    return pl.pallas_call(
        body,
        grid=grid,
        in_specs=[pl.BlockSpec((_NC, _BN, _NFEAT), lambda i: (0, i, 0)),
                  pl.BlockSpec((_NC, _BN, _AH), lambda i: (0, i, 0)),
                  pl.BlockSpec((1, _NFEAT), lambda i: (0, 0)),
                  pl.BlockSpec((_AH, _NFEAT), lambda i: (0, 0)),
                  pl.BlockSpec((_NFEAT, _NCLASS), lambda i: (0, 0)),
                  pl.BlockSpec((_NCLASS, _AH), lambda i: (0, 0)),
                  pl.BlockSpec((_NCLASS, _AH), lambda i: (0, 0))],
        out_specs=[pl.BlockSpec((_BN, _NCLASS), lambda i: (i, 0)),
                   pl.BlockSpec((_BN, _AH), lambda i: (i, 0)),
                   pl.BlockSpec((_BN, _AH), lambda i: (i, 0))],
        out_shape=[jax.ShapeDtypeStruct((_N, _NCLASS), jnp.float32),
                   jax.ShapeDtypeStruct((_N, _AH), jnp.float32),
                   jax.ShapeDtypeStruct((_N, _AH), jnp.float32)],
    )(num1, den1, b1r, EXPAND, W2f, A2s, A2d)


def _tc_post(num2, den2):
    grid = (_NP // _BN,)

    def body(n_ref, d_ref, o_ref):
        num = n_ref[0] + n_ref[1]
        den = d_ref[0, :, 0:1] + d_ref[1, :, 0:1]
        o_ref[...] = num / (den + 1e-16)

    return pl.pallas_call(
        body,
        grid=grid,
        in_specs=[pl.BlockSpec((_NC, _BN, _NCLASS), lambda i: (0, i, 0)),
                  pl.BlockSpec((_NC, _BN, _AH), lambda i: (0, i, 0))],
        out_specs=pl.BlockSpec((_BN, _NCLASS), lambda i: (i, 0)),
        out_shape=jax.ShapeDtypeStruct((_N, _NCLASS), jnp.float32),
    )(num2, den2)


def kernel(x, adj, W1, a1_src, a1_dst, b1, W2, a2_src, a2_dst):
    W1f = W1.transpose(1, 0, 2).reshape(_NFEAT, _NHEAD * _NHID)
    eye = jnp.eye(_NHEAD, dtype=jnp.float32)
    EXPAND = jnp.repeat(eye, _NHID, axis=1)
    sel = EXPAND.T
    A1s = sel * a1_src.reshape(-1)[:, None]
    A1d = sel * a1_dst.reshape(-1)[:, None]
    b1r = b1.reshape(1, _NFEAT)
    W2f = W2[0]
    pad = jnp.zeros((_NCLASS, _AH - 1), jnp.float32)
    A2s = jnp.concatenate([a2_src[0][:, None], pad], axis=1)
    A2d = jnp.concatenate([a2_dst[0][:, None], pad], axis=1)

    h1, oas1, oad1 = _tc_layer1(x, W1f, A1s, A1d)
    num1, den1 = _sc_edge_pass(_NHEAD, _NHEAD * _NHID)(adj, h1, oas1, oad1)
    h2, oas2, oad2 = _tc_mid(num1, den1, b1r, EXPAND, W2f, A2s, A2d)
    num2, den2 = _sc_edge_pass(1, _NCLASS)(adj, h2, oas2, oad2)
    return _tc_post(num2, den2)


# speedup vs baseline: 54.7640x; 54.7640x over previous
import jax
import jax.numpy as jnp
from jax import lax
from jax.experimental import pallas as pl
from jax.experimental.pallas import tpu as pltpu
from jax.experimental.pallas import tpu_sc as plsc

_N = 10000
_E = 320000
_NFEAT = 128
_NHID = 16
_NHEAD = 8
_NCLASS = 64

_NC = 2
_NS = 16
_NW = _NC * _NS
_L = 16
_CH = 128
_NP = 10240
_AH = 8
_BN = 256


def _sc_edge_pass(H, HO):
    n_chunks = _E // _CH
    O = HO // H
    nz = _NP // _CH // _NS

    mesh = plsc.VectorSubcoreMesh(core_axis_name="c", subcore_axis_name="s",
                                  num_cores=_NC, num_subcores=_NS)

    def body(adj, hmat, asrc, adst, num_out, den_out,
             sidx, didx, hrows, asr, adr, wem, num_s, den_s, sem):
        c = lax.axis_index("c")
        s = lax.axis_index("s")
        wid = s * _NC + c
        zero16 = jnp.zeros((_L,), jnp.float32)
        iota = lax.iota(jnp.int32, _L)

        def zrow(k, carry):
            for j in range(HO // _L):
                hrows[k, pl.ds(j * _L, _L)] = zero16
            return carry
        lax.fori_loop(0, _CH, zrow, 0)
        r_pat = lax.shift_right_logical(iota, 3)
        c_pat = lax.bitwise_and(iota, 7)
        for i in range(_CH // 2):
            plsc.store_scatter(wem, [r_pat + 2 * i, c_pat], zero16)
        for j in range(nz):
            r = (s * nz + j) * _CH
            pltpu.sync_copy(hrows, num_s.at[pl.ds(r, _CH)])
            pltpu.sync_copy(wem, den_s.at[pl.ds(r, _CH)])
        plsc.subcore_barrier()

        cols = [jnp.full((_L,), h, jnp.int32) for h in range(H)]
        hpat = lax.bitwise_and(iota, H - 1)

        def chunk(i, carry):
            base = (wid + _NW * i) * _CH
            pltpu.sync_copy(adj.at[0, pl.ds(base, _CH)], sidx)
            pltpu.sync_copy(adj.at[1, pl.ds(base, _CH)], didx)
            cp1 = pltpu.async_copy(hmat.at[sidx], hrows, sem)
            cp2 = pltpu.async_copy(asrc.at[sidx], asr, sem)
            cp3 = pltpu.async_copy(adst.at[didx], adr, sem)
            cp1.wait()
            cp2.wait()
            cp3.wait()
            for g in range(_CH // _L):
                ridx = iota + (g * _L)
                for h in range(H):
                    ev = (plsc.load_gather(asr, [ridx, cols[h]])
                          + plsc.load_gather(adr, [ridx, cols[h]]))
                    ev = jnp.maximum(ev, 0.2 * ev)
                    plsc.store_scatter(wem, [ridx, cols[h]], jnp.exp(ev))
            def mul_body(k, carry2):
                kvec = jnp.zeros((_L,), jnp.int32) + k
                wexp = plsc.load_gather(wem, [kvec, hpat])
                for j in range(HO // _L):
                    sl = pl.ds(j * _L, _L)
                    hrows[k, sl] = hrows[k, sl] * wexp
                return carry2
            lax.fori_loop(0, _CH, mul_body, 0)
            pltpu.sync_copy(hrows, num_s.at[didx], add=True)
            pltpu.sync_copy(wem, den_s.at[didx], add=True)
            return carry

        n_my = (n_chunks - wid + _NW - 1) // _NW
        lax.fori_loop(0, n_my, chunk, 0)

        plsc.subcore_barrier()
        for j in range(nz):
            r = (s * nz + j) * _CH
            pltpu.sync_copy(num_s.at[pl.ds(r, _CH)], hrows)
            pltpu.sync_copy(hrows, num_out.at[c, pl.ds(r, _CH)])
            pltpu.sync_copy(den_s.at[pl.ds(r, _CH)], wem)
            pltpu.sync_copy(wem, den_out.at[c, pl.ds(r, _CH)])

    return pl.kernel(
        body,
        out_type=(jax.ShapeDtypeStruct((_NC, _NP, HO), jnp.float32),
                  jax.ShapeDtypeStruct((_NC, _NP, _AH), jnp.float32)),
        mesh=mesh,
        compiler_params=pltpu.CompilerParams(needs_layout_passes=False,
                                             use_tc_tiling_on_sc=False),
        scratch_types=[
            pltpu.VMEM((_CH,), jnp.int32),
            pltpu.VMEM((_CH,), jnp.int32),
            pltpu.VMEM((_CH, HO), jnp.float32),
            pltpu.VMEM((_CH, _AH), jnp.float32),
            pltpu.VMEM((_CH, _AH), jnp.float32),
            pltpu.VMEM((_CH, _AH), jnp.float32),
            pltpu.VMEM_SHARED((_NP, HO), jnp.float32),
            pltpu.VMEM_SHARED((_NP, _AH), jnp.float32),
            pltpu.SemaphoreType.DMA,
        ],
    )


def _tc_layer1(x, W1f, A1s, A1d):
    grid = ((_N + _BN - 1) // _BN,)

    def body(x_ref, w_ref, as_ref, ad_ref, h_ref, oas_ref, oad_ref):
        h = jnp.dot(x_ref[...], w_ref[...], preferred_element_type=jnp.float32)
        h_ref[...] = h
        oas_ref[...] = jnp.dot(h, as_ref[...], preferred_element_type=jnp.float32)
        oad_ref[...] = jnp.dot(h, ad_ref[...], preferred_element_type=jnp.float32)

    return pl.pallas_call(
        body,
        grid=grid,
        in_specs=[pl.BlockSpec((_BN, _NFEAT), lambda i: (i, 0)),
                  pl.BlockSpec((_NFEAT, _NFEAT), lambda i: (0, 0)),
                  pl.BlockSpec((_NFEAT, _AH), lambda i: (0, 0)),
                  pl.BlockSpec((_NFEAT, _AH), lambda i: (0, 0))],
        out_specs=[pl.BlockSpec((_BN, _NFEAT), lambda i: (i, 0)),
                   pl.BlockSpec((_BN, _AH), lambda i: (i, 0)),
                   pl.BlockSpec((_BN, _AH), lambda i: (i, 0))],
        out_shape=[jax.ShapeDtypeStruct((_N, _NFEAT), jnp.float32),
                   jax.ShapeDtypeStruct((_N, _AH), jnp.float32),
                   jax.ShapeDtypeStruct((_N, _AH), jnp.float32)],
    )(x, W1f, A1s, A1d)


def _tc_mid(num1, den1, b1r, EXPAND, W2f, A2s, A2d):
    grid = (_NP // _BN,)

    def body(n_ref, d_ref, b_ref, e_ref, w_ref, as_ref, ad_ref,
             h2_ref, oas_ref, oad_ref):
        num = n_ref[0] + n_ref[1]
        den = d_ref[0] + d_ref[1]
        den_w = jnp.dot(den, e_ref[...], preferred_element_type=jnp.float32)
        out1 = num / (den_w + 1e-16) + b_ref[...]
        h1 = jnp.where(out1 > 0, out1, jnp.exp(jnp.minimum(out1, 0.0)) - 1.0)
        h2 = jnp.dot(h1, w_ref[...], preferred_element_type=jnp.float32)
        h2_ref[...] = h2
        oas_ref[...] = jnp.dot(h2, as_ref[...], preferred_element_type=jnp.float32)
        oad_ref[...] = jnp.dot(h2, ad_ref[...], preferred_element_type=jnp.float32)

    return pl.pallas_call(
        body,
        grid=grid,
        in_specs=[pl.BlockSpec((_NC, _BN, _NFEAT), lambda i: (0, i, 0)),
                  pl.BlockSpec((_NC, _BN, _AH), lambda i: (0, i, 0)),
                  pl.BlockSpec((1, _NFEAT), lambda i: (0, 0)),
                  pl.BlockSpec((_AH, _NFEAT), lambda i: (0, 0)),
                  pl.BlockSpec((_NFEAT, _NCLASS), lambda i: (0, 0)),
                  pl.BlockSpec((_NCLASS, _AH), lambda i: (0, 0)),
                  pl.BlockSpec((_NCLASS, _AH), lambda i: (0, 0))],
        out_specs=[pl.BlockSpec((_BN, _NCLASS), lambda i: (i, 0)),
                   pl.BlockSpec((_BN, _AH), lambda i: (i, 0)),
                   pl.BlockSpec((_BN, _AH), lambda i: (i, 0))],
        out_shape=[jax.ShapeDtypeStruct((_N, _NCLASS), jnp.float32),
                   jax.ShapeDtypeStruct((_N, _AH), jnp.float32),
                   jax.ShapeDtypeStruct((_N, _AH), jnp.float32)],
    )(num1, den1, b1r, EXPAND, W2f, A2s, A2d)


def _tc_post(num2, den2):
    grid = (_NP // _BN,)

    def body(n_ref, d_ref, o_ref):
        num = n_ref[0] + n_ref[1]
        den = d_ref[0, :, 0:1] + d_ref[1, :, 0:1]
        o_ref[...] = num / (den + 1e-16)

    return pl.pallas_call(
        body,
        grid=grid,
        in_specs=[pl.BlockSpec((_NC, _BN, _NCLASS), lambda i: (0, i, 0)),
                  pl.BlockSpec((_NC, _BN, _AH), lambda i: (0, i, 0))],
        out_specs=pl.BlockSpec((_BN, _NCLASS), lambda i: (i, 0)),
        out_shape=jax.ShapeDtypeStruct((_N, _NCLASS), jnp.float32),
    )(num2, den2)


def kernel(x, adj, W1, a1_src, a1_dst, b1, W2, a2_src, a2_dst):
    W1f = W1.transpose(1, 2, 0).reshape(_NFEAT, _NHEAD * _NHID)
    eye = jnp.eye(_NHEAD, dtype=jnp.float32)
    EXPAND = jnp.tile(eye, (1, _NHID))
    sel = EXPAND.T
    A1s = sel * a1_src.T.reshape(-1)[:, None]
    A1d = sel * a1_dst.T.reshape(-1)[:, None]
    b1r = b1.reshape(_NHEAD, _NHID).T.reshape(1, _NFEAT)
    W2f = W2[0].reshape(_NHEAD, _NHID, _NCLASS).transpose(1, 0, 2)
    W2f = W2f.reshape(_NFEAT, _NCLASS)
    pad = jnp.zeros((_NCLASS, _AH - 1), jnp.float32)
    A2s = jnp.concatenate([a2_src[0][:, None], pad], axis=1)
    A2d = jnp.concatenate([a2_dst[0][:, None], pad], axis=1)

    h1, oas1, oad1 = _tc_layer1(x, W1f, A1s, A1d)
    num1, den1 = _sc_edge_pass(_NHEAD, _NHEAD * _NHID)(adj, h1, oas1, oad1)
    h2, oas2, oad2 = _tc_mid(num1, den1, b1r, EXPAND, W2f, A2s, A2d)
    num2, den2 = _sc_edge_pass(1, _NCLASS)(adj, h2, oas2, oad2)
    return _tc_post(num2, den2)


# speedup vs baseline: 77.9154x; 1.4227x over previous
import jax
import jax.numpy as jnp
from jax import lax
from jax.experimental import pallas as pl
from jax.experimental.pallas import tpu as pltpu
from jax.experimental.pallas import tpu_sc as plsc

_N = 10000
_E = 320000
_NFEAT = 128
_NHID = 16
_NHEAD = 8
_NCLASS = 64

_NC = 2
_NS = 16
_NW = _NC * _NS
_L = 16
_CH = 128
_NP = 10240
_AH = 8
_BN = 256


def _sc_edge_pass(H, HO):
    n_chunks = _E // _CH
    O = HO // H
    nz = _NP // _CH // _NS

    mesh = plsc.VectorSubcoreMesh(core_axis_name="c", subcore_axis_name="s",
                                  num_cores=_NC, num_subcores=_NS)

    def body(adj, hmat, asrc, adst, num_out, den_out,
             ij0, ij1, hrows0, hrows1, asr0, asr1, adr0, adr1, wem0, wem1,
             num_s, den_s, sem0, sem1):
        c = lax.axis_index("c")
        s = lax.axis_index("s")
        wid = s * _NC + c
        zero16 = jnp.zeros((_L,), jnp.float32)
        iota = lax.iota(jnp.int32, _L)

        def zrow(k, carry):
            for j in range(HO // _L):
                hrows0[k, pl.ds(j * _L, _L)] = zero16
            return carry
        lax.fori_loop(0, _CH, zrow, 0)
        r_pat = lax.shift_right_logical(iota, 3)
        c_pat = lax.bitwise_and(iota, 7)
        for i in range(_CH // 2):
            plsc.store_scatter(wem0, [r_pat + 2 * i, c_pat], zero16)
            plsc.store_scatter(wem1, [r_pat + 2 * i, c_pat], zero16)
        for j in range(nz):
            r = (s * nz + j) * _CH
            pltpu.sync_copy(hrows0, num_s.at[pl.ds(r, _CH)])
            pltpu.sync_copy(wem0, den_s.at[pl.ds(r, _CH)])
        plsc.subcore_barrier()

        cols = [jnp.full((_L,), h, jnp.int32) for h in range(H)]
        hpat = lax.bitwise_and(iota, H - 1)
        bufs = ((ij0, hrows0, asr0, adr0, wem0, sem0),
                (ij1, hrows1, asr1, adr1, wem1, sem1))

        def issue(i, b):
            ij, hrows, asr, adr, _, sem = bufs[b]
            base = (wid + _NW * i) * _CH
            pltpu.sync_copy(adj.at[:, pl.ds(base, _CH)], ij)
            pltpu.async_copy(hmat.at[ij.at[0]], hrows, sem)
            pltpu.async_copy(asrc.at[ij.at[0]], asr, sem)
            pltpu.async_copy(adst.at[ij.at[1]], adr, sem)

        def process(b):
            ij, hrows, asr, adr, wem, sem = bufs[b]
            pltpu.make_async_copy(hmat.at[ij.at[0]], hrows, sem).wait()
            pltpu.make_async_copy(asrc.at[ij.at[0]], asr, sem).wait()
            pltpu.make_async_copy(adst.at[ij.at[1]], adr, sem).wait()
            for g in range(_CH // _L):
                ridx = iota + (g * _L)
                for h in range(H):
                    ev = (plsc.load_gather(asr, [ridx, cols[h]])
                          + plsc.load_gather(adr, [ridx, cols[h]]))
                    ev = jnp.maximum(ev, 0.2 * ev)
                    plsc.store_scatter(wem, [ridx, cols[h]], jnp.exp(ev))
            def mul_body(k, carry2):
                kvec = jnp.zeros((_L,), jnp.int32) + k
                wexp = plsc.load_gather(wem, [kvec, hpat])
                for j in range(HO // _L):
                    sl = pl.ds(j * _L, _L)
                    hrows[k, sl] = hrows[k, sl] * wexp
                return carry2
            lax.fori_loop(0, _CH, mul_body, 0)
            pltpu.sync_copy(hrows, num_s.at[ij.at[1]], add=True)
            pltpu.sync_copy(wem, den_s.at[ij.at[1]], add=True)

        n_my = (n_chunks - wid + _NW - 1) // _NW
        n_pairs = (n_my + 1) // 2
        issue(0, 0)

        def pair(j, carry):
            i1 = 2 * j + 1
            i2 = 2 * j + 2

            @pl.when(i1 < n_my)
            def _():
                issue(i1, 1)
            process(0)

            @pl.when(i2 < n_my)
            def _():
                issue(i2, 0)

            @pl.when(i1 < n_my)
            def _():
                process(1)
            return carry

        lax.fori_loop(0, n_pairs, pair, 0)

        plsc.subcore_barrier()
        for j in range(nz):
            r = (s * nz + j) * _CH
            pltpu.sync_copy(num_s.at[pl.ds(r, _CH)], hrows0)
            pltpu.sync_copy(hrows0, num_out.at[c, pl.ds(r, _CH)])
            pltpu.sync_copy(den_s.at[pl.ds(r, _CH)], wem0)
            pltpu.sync_copy(wem0, den_out.at[c, pl.ds(r, _CH)])

    return pl.kernel(
        body,
        out_type=(jax.ShapeDtypeStruct((_NC, _NP, HO), jnp.float32),
                  jax.ShapeDtypeStruct((_NC, _NP, _AH), jnp.float32)),
        mesh=mesh,
        compiler_params=pltpu.CompilerParams(needs_layout_passes=False,
                                             use_tc_tiling_on_sc=False),
        scratch_types=[
            pltpu.VMEM((2, _CH), jnp.int32),
            pltpu.VMEM((2, _CH), jnp.int32),
            pltpu.VMEM((_CH, HO), jnp.float32),
            pltpu.VMEM((_CH, HO), jnp.float32),
            pltpu.VMEM((_CH, _AH), jnp.float32),
            pltpu.VMEM((_CH, _AH), jnp.float32),
            pltpu.VMEM((_CH, _AH), jnp.float32),
            pltpu.VMEM((_CH, _AH), jnp.float32),
            pltpu.VMEM((_CH, _AH), jnp.float32),
            pltpu.VMEM((_CH, _AH), jnp.float32),
            pltpu.VMEM_SHARED((_NP, HO), jnp.float32),
            pltpu.VMEM_SHARED((_NP, _AH), jnp.float32),
            pltpu.SemaphoreType.DMA,
            pltpu.SemaphoreType.DMA,
        ],
    )


def _tc_layer1(x, W1f, A1s, A1d):
    grid = ((_N + _BN - 1) // _BN,)

    def body(x_ref, w_ref, as_ref, ad_ref, h_ref, oas_ref, oad_ref):
        h = jnp.dot(x_ref[...], w_ref[...], preferred_element_type=jnp.float32)
        h_ref[...] = h
        oas_ref[...] = jnp.dot(h, as_ref[...], preferred_element_type=jnp.float32)
        oad_ref[...] = jnp.dot(h, ad_ref[...], preferred_element_type=jnp.float32)

    return pl.pallas_call(
        body,
        grid=grid,
        in_specs=[pl.BlockSpec((_BN, _NFEAT), lambda i: (i, 0)),
                  pl.BlockSpec((_NFEAT, _NFEAT), lambda i: (0, 0)),
                  pl.BlockSpec((_NFEAT, _AH), lambda i: (0, 0)),
                  pl.BlockSpec((_NFEAT, _AH), lambda i: (0, 0))],
        out_specs=[pl.BlockSpec((_BN, _NFEAT), lambda i: (i, 0)),
                   pl.BlockSpec((_BN, _AH), lambda i: (i, 0)),
                   pl.BlockSpec((_BN, _AH), lambda i: (i, 0))],
        out_shape=[jax.ShapeDtypeStruct((_N, _NFEAT), jnp.float32),
                   jax.ShapeDtypeStruct((_N, _AH), jnp.float32),
                   jax.ShapeDtypeStruct((_N, _AH), jnp.float32)],
    )(x, W1f, A1s, A1d)


def _tc_mid(num1, den1, b1r, EXPAND, W2f, A2s, A2d):
    grid = (_NP // _BN,)

    def body(n_ref, d_ref, b_ref, e_ref, w_ref, as_ref, ad_ref,
             h2_ref, oas_ref, oad_ref):
        num = n_ref[0] + n_ref[1]
        den = d_ref[0] + d_ref[1]
        den_w = jnp.dot(den, e_ref[...], preferred_element_type=jnp.float32)
        out1 = num / (den_w + 1e-16) + b_ref[...]
        h1 = jnp.where(out1 > 0, out1, jnp.exp(jnp.minimum(out1, 0.0)) - 1.0)
        h2 = jnp.dot(h1, w_ref[...], preferred_element_type=jnp.float32)
        h2_ref[...] = h2
        oas_ref[...] = jnp.dot(h2, as_ref[...], preferred_element_type=jnp.float32)
        oad_ref[...] = jnp.dot(h2, ad_ref[...], preferred_element_type=jnp.float32)

    return pl.pallas_call(
        body,
        grid=grid,
        in_specs=[pl.BlockSpec((_NC, _BN, _NFEAT), lambda i: (0, i, 0)),
                  pl.BlockSpec((_NC, _BN, _AH), lambda i: (0, i, 0)),
                  pl.BlockSpec((1, _NFEAT), lambda i: (0, 0)),
                  pl.BlockSpec((_AH, _NFEAT), lambda i: (0, 0)),
                  pl.BlockSpec((_NFEAT, _NCLASS), lambda i: (0, 0)),
                  pl.BlockSpec((_NCLASS, _AH), lambda i: (0, 0)),
                  pl.BlockSpec((_NCLASS, _AH), lambda i: (0, 0))],
        out_specs=[pl.BlockSpec((_BN, _NCLASS), lambda i: (i, 0)),
                   pl.BlockSpec((_BN, _AH), lambda i: (i, 0)),
                   pl.BlockSpec((_BN, _AH), lambda i: (i, 0))],
        out_shape=[jax.ShapeDtypeStruct((_N, _NCLASS), jnp.float32),
                   jax.ShapeDtypeStruct((_N, _AH), jnp.float32),
                   jax.ShapeDtypeStruct((_N, _AH), jnp.float32)],
    )(num1, den1, b1r, EXPAND, W2f, A2s, A2d)


def _tc_post(num2, den2):
    grid = (_NP // _BN,)

    def body(n_ref, d_ref, o_ref):
        num = n_ref[0] + n_ref[1]
        den = d_ref[0, :, 0:1] + d_ref[1, :, 0:1]
        o_ref[...] = num / (den + 1e-16)

    return pl.pallas_call(
        body,
        grid=grid,
        in_specs=[pl.BlockSpec((_NC, _BN, _NCLASS), lambda i: (0, i, 0)),
                  pl.BlockSpec((_NC, _BN, _AH), lambda i: (0, i, 0))],
        out_specs=pl.BlockSpec((_BN, _NCLASS), lambda i: (i, 0)),
        out_shape=jax.ShapeDtypeStruct((_N, _NCLASS), jnp.float32),
    )(num2, den2)


def kernel(x, adj, W1, a1_src, a1_dst, b1, W2, a2_src, a2_dst):
    W1f = W1.transpose(1, 2, 0).reshape(_NFEAT, _NHEAD * _NHID)
    eye = jnp.eye(_NHEAD, dtype=jnp.float32)
    EXPAND = jnp.tile(eye, (1, _NHID))
    sel = EXPAND.T
    A1s = sel * a1_src.T.reshape(-1)[:, None]
    A1d = sel * a1_dst.T.reshape(-1)[:, None]
    b1r = b1.reshape(_NHEAD, _NHID).T.reshape(1, _NFEAT)
    W2f = W2[0].reshape(_NHEAD, _NHID, _NCLASS).transpose(1, 0, 2)
    W2f = W2f.reshape(_NFEAT, _NCLASS)
    pad = jnp.zeros((_NCLASS, _AH - 1), jnp.float32)
    A2s = jnp.concatenate([a2_src[0][:, None], pad], axis=1)
    A2d = jnp.concatenate([a2_dst[0][:, None], pad], axis=1)

    h1, oas1, oad1 = _tc_layer1(x, W1f, A1s, A1d)
    num1, den1 = _sc_edge_pass(_NHEAD, _NHEAD * _NHID)(adj, h1, oas1, oad1)
    h2, oas2, oad2 = _tc_mid(num1, den1, b1r, EXPAND, W2f, A2s, A2d)
    num2, den2 = _sc_edge_pass(1, _NCLASS)(adj, h2, oas2, oad2)
    return _tc_post(num2, den2)


# speedup vs baseline: 114.3856x; 1.4681x over previous
import jax
import jax.numpy as jnp
import numpy as np
from jax import lax
from jax.experimental import pallas as pl
from jax.experimental.pallas import tpu as pltpu
from jax.experimental.pallas import tpu_sc as plsc

_N = 10000
_E = 320000
_NFEAT = 128
_NHID = 16
_NHEAD = 8
_NCLASS = 64

_NC = 2
_NS = 16
_NW = _NC * _NS
_L = 16
_CH = 128
_NP = 10240
_AH = 8
_BN = 1024


def _interleave_perm(width):
    return np.arange(width).reshape(-1, 2, 16).transpose(0, 2, 1).reshape(-1)


_P1 = _interleave_perm(_NFEAT)
_P2 = _interleave_perm(_NCLASS)


def _sc_edge_pass(H, HO, CH, use_bf16):
    n_chunks = _E // CH
    O = HO // H
    nz = _NP // CH // _NS

    mesh = plsc.VectorSubcoreMesh(core_axis_name="c", subcore_axis_name="s",
                                  num_cores=_NC, num_subcores=_NS)

    def body(adj, hmat, asrc, adst, num_out, den_out,
             ij0, ij1, ij2, hbf0, hbf1, hbf2, hrows0, hrows1, hrows2,
             asr0, asr1, asr2, adr0, adr1, adr2, wem0, wem1, wem2,
             num_s, den_s, gsem0, gsem1, gsem2, ssem0, ssem1, ssem2):
        c = lax.axis_index("c")
        s = lax.axis_index("s")
        wid = s * _NC + c
        zero16 = jnp.zeros((_L,), jnp.float32)
        iota = lax.iota(jnp.int32, _L)

        def zrow(k, carry):
            for j in range(HO // _L):
                hrows0[k, pl.ds(j * _L, _L)] = zero16
            return carry
        lax.fori_loop(0, CH, zrow, 0)
        r_pat = lax.shift_right_logical(iota, 3)
        c_pat = lax.bitwise_and(iota, 7)
        for i in range(CH // 2):
            plsc.store_scatter(wem0, [r_pat + 2 * i, c_pat], zero16)
            plsc.store_scatter(wem1, [r_pat + 2 * i, c_pat], zero16)
            plsc.store_scatter(wem2, [r_pat + 2 * i, c_pat], zero16)
        for j in range(nz):
            r = (s * nz + j) * CH
            pltpu.sync_copy(hrows0, num_s.at[pl.ds(r, CH)])
            pltpu.sync_copy(wem0, den_s.at[pl.ds(r, CH)])

        cols = [jnp.full((_L,), h, jnp.int32) for h in range(H)]
        hpat = lax.bitwise_and(iota, H - 1)
        bufs = ((ij0, hbf0, hrows0, asr0, adr0, wem0, gsem0, ssem0),
                (ij1, hbf1, hrows1, asr1, adr1, wem1, gsem1, ssem1),
                (ij2, hbf2, hrows2, asr2, adr2, wem2, gsem2, ssem2))

        def issue(i, b):
            ij, hbf, hrows, asr, adr, wem, gsem, ssem = bufs[b]

            @pl.when(i >= 3)
            def _():
                pltpu.make_async_copy(hrows, num_s.at[ij.at[1]], ssem).wait()
                pltpu.make_async_copy(wem, den_s.at[ij.at[1]], ssem).wait()

            base = (wid + _NW * i) * CH
            pltpu.sync_copy(adj.at[:, pl.ds(base, CH)], ij)
            pltpu.async_copy(hmat.at[ij.at[0]], hbf if use_bf16 else hrows, gsem)
            pltpu.async_copy(asrc.at[ij.at[0]], asr, gsem)
            pltpu.async_copy(adst.at[ij.at[1]], adr, gsem)

        def process(b):
            ij, hbf, hrows, asr, adr, wem, gsem, ssem = bufs[b]
            pltpu.make_async_copy(hmat.at[ij.at[0]],
                                  hbf if use_bf16 else hrows, gsem).wait()
            pltpu.make_async_copy(asrc.at[ij.at[0]], asr, gsem).wait()
            pltpu.make_async_copy(adst.at[ij.at[1]], adr, gsem).wait()
            for g in range(CH // _L):
                ridx = iota + (g * _L)
                for h in range(H):
                    ev = (plsc.load_gather(asr, [ridx, cols[h]])
                          + plsc.load_gather(adr, [ridx, cols[h]]))
                    ev = jnp.maximum(ev, 0.2 * ev)
                    plsc.store_scatter(wem, [ridx, cols[h]], jnp.exp(ev))

            @plsc.parallel_loop(0, CH, unroll=2)
            def mul_body(k):
                kvec = jnp.zeros((_L,), jnp.int32) + k
                wexp = plsc.load_gather(wem, [kvec, hpat])
                if use_bf16:
                    for j in range(HO // (2 * _L)):
                        hb = hbf[k, pl.ds(2 * _L * j, 2 * _L)]
                        va, vb = plsc.unpack(hb,
                                             format=plsc.PackFormat.INTERLEAVED)
                        hrows[k, pl.ds(2 * _L * j, _L)] = va * wexp
                        hrows[k, pl.ds(2 * _L * j + _L, _L)] = vb * wexp
                else:
                    for j in range(HO // _L):
                        sl = pl.ds(j * _L, _L)
                        hrows[k, sl] = hrows[k, sl] * wexp

            pltpu.async_copy(hrows, num_s.at[ij.at[1]], ssem, add=True)
            pltpu.async_copy(wem, den_s.at[ij.at[1]], ssem, add=True)

        n_my = (n_chunks - wid + _NW - 1) // _NW
        n_tri = (n_my + 2) // 3

        issue(0, 0)
        issue(1, 1)
        plsc.subcore_barrier()

        def triple(j, carry):
            i0 = 3 * j
            for t in range(3):
                it = i0 + t

                @pl.when(it < n_my)
                def _(t=t):
                    process(t)

                @pl.when(it + 2 < n_my)
                def _(t=t, it=it):
                    issue(it + 2, (t + 2) % 3)
            return carry

        lax.fori_loop(0, n_tri, triple, 0)

        for b in range(3):
            @pl.when(n_my > b)
            def _(b=b):
                ij, _hbf, hrows, _asr, _adr, wem, _g, ssem = bufs[b]
                pltpu.make_async_copy(hrows, num_s.at[ij.at[1]], ssem).wait()
                pltpu.make_async_copy(wem, den_s.at[ij.at[1]], ssem).wait()

        plsc.subcore_barrier()
        for j in range(nz):
            r = (s * nz + j) * CH
            pltpu.sync_copy(num_s.at[pl.ds(r, CH)], hrows0)
            pltpu.sync_copy(hrows0, num_out.at[c, pl.ds(r, CH)])
            pltpu.sync_copy(den_s.at[pl.ds(r, CH)], wem0)
            pltpu.sync_copy(wem0, den_out.at[c, pl.ds(r, CH)])

    return pl.kernel(
        body,
        out_type=(jax.ShapeDtypeStruct((_NC, _NP, HO), jnp.float32),
                  jax.ShapeDtypeStruct((_NC, _NP, _AH), jnp.float32)),
        mesh=mesh,
        compiler_params=pltpu.CompilerParams(needs_layout_passes=False,
                                             use_tc_tiling_on_sc=False),
        scratch_types=(
            [pltpu.VMEM((2, CH), jnp.int32)] * 3
            + [pltpu.VMEM((CH, HO) if use_bf16 else (2, _L),
                          jnp.bfloat16)] * 3
            + [pltpu.VMEM((CH, HO), jnp.float32)] * 3
            + [pltpu.VMEM((CH, _AH), jnp.float32)] * 6
            + [pltpu.VMEM((CH, _AH), jnp.float32)] * 3
            + [pltpu.VMEM_SHARED((_NP, HO), jnp.float32),
               pltpu.VMEM_SHARED((_NP, _AH), jnp.float32)]
            + [pltpu.SemaphoreType.DMA] * 6
        ),
    )


def _tc_layer1(x, W1f, A1):
    grid = ((_N + _BN - 1) // _BN,)

    def body(x_ref, w_ref, a_ref, h_ref, oa_ref):
        h = jnp.dot(x_ref[...], w_ref[...], preferred_element_type=jnp.float32)
        h_ref[...] = h
        oa_ref[...] = jnp.dot(h, a_ref[...], preferred_element_type=jnp.float32)

    return pl.pallas_call(
        body,
        grid=grid,
        in_specs=[pl.BlockSpec((_BN, _NFEAT), lambda i: (i, 0)),
                  pl.BlockSpec((_NFEAT, _NFEAT), lambda i: (0, 0)),
                  pl.BlockSpec((_NFEAT, 2 * _AH), lambda i: (0, 0))],
        out_specs=[pl.BlockSpec((_BN, _NFEAT), lambda i: (i, 0)),
                   pl.BlockSpec((_BN, 2 * _AH), lambda i: (i, 0))],
        out_shape=[jax.ShapeDtypeStruct((_N, _NFEAT), jnp.float32),
                   jax.ShapeDtypeStruct((_N, 2 * _AH), jnp.float32)],
    )(x, W1f, A1)


def _tc_mid(num1, den1, b1r, EXPAND, W2f, A2):
    grid = (_NP // _BN,)

    def body(n_ref, d_ref, b_ref, e_ref, w_ref, a_ref, h2_ref, oa_ref):
        num = n_ref[0] + n_ref[1]
        den = d_ref[0] + d_ref[1]
        den_w = jnp.dot(den, e_ref[...], preferred_element_type=jnp.float32)
        out1 = num / (den_w + 1e-16) + b_ref[...]
        h1 = jnp.where(out1 > 0, out1, jnp.exp(jnp.minimum(out1, 0.0)) - 1.0)
        h2 = jnp.dot(h1, w_ref[...], preferred_element_type=jnp.float32)
        h2_ref[...] = h2.astype(jnp.bfloat16)
        oa_ref[...] = jnp.dot(h2, a_ref[...], preferred_element_type=jnp.float32)

    return pl.pallas_call(
        body,
        grid=grid,
        in_specs=[pl.BlockSpec((_NC, _BN, _NFEAT), lambda i: (0, i, 0)),
                  pl.BlockSpec((_NC, _BN, _AH), lambda i: (0, i, 0)),
                  pl.BlockSpec((1, _NFEAT), lambda i: (0, 0)),
                  pl.BlockSpec((_AH, _NFEAT), lambda i: (0, 0)),
                  pl.BlockSpec((_NFEAT, _NCLASS), lambda i: (0, 0)),
                  pl.BlockSpec((_NCLASS, 2 * _AH), lambda i: (0, 0))],
        out_specs=[pl.BlockSpec((_BN, _NCLASS), lambda i: (i, 0)),
                   pl.BlockSpec((_BN, 2 * _AH), lambda i: (i, 0))],
        out_shape=[jax.ShapeDtypeStruct((_N, _NCLASS), jnp.bfloat16),
                   jax.ShapeDtypeStruct((_N, 2 * _AH), jnp.float32)],
    )(num1, den1, b1r, EXPAND, W2f, A2)


def _tc_post(num2, den2):
    grid = (_NP // _BN,)

    def body(n_ref, d_ref, o_ref):
        num = n_ref[0] + n_ref[1]
        den = d_ref[0, :, 0:1] + d_ref[1, :, 0:1]
        o_ref[...] = num / (den + 1e-16)

    return pl.pallas_call(
        body,
        grid=grid,
        in_specs=[pl.BlockSpec((_NC, _BN, _NCLASS), lambda i: (0, i, 0)),
                  pl.BlockSpec((_NC, _BN, _AH), lambda i: (0, i, 0))],
        out_specs=pl.BlockSpec((_BN, _NCLASS), lambda i: (i, 0)),
        out_shape=jax.ShapeDtypeStruct((_N, _NCLASS), jnp.float32),
    )(num2, den2)


def kernel(x, adj, W1, a1_src, a1_dst, b1, W2, a2_src, a2_dst):
    W1f = W1.transpose(1, 2, 0).reshape(_NFEAT, _NHEAD * _NHID)
    eye = jnp.eye(_NHEAD, dtype=jnp.float32)
    EXPAND = jnp.tile(eye, (1, _NHID))
    sel = EXPAND.T
    A1 = jnp.concatenate([sel * a1_src.T.reshape(-1)[:, None],
                          sel * a1_dst.T.reshape(-1)[:, None]], axis=1)
    b1r = b1.reshape(_NHEAD, _NHID).T.reshape(1, _NFEAT)
    W2f = W2[0].reshape(_NHEAD, _NHID, _NCLASS).transpose(1, 0, 2)
    W2f = W2f.reshape(_NFEAT, _NCLASS)[:, _P2]
    pad = jnp.zeros((_NCLASS, _AH - 1), jnp.float32)
    A2 = jnp.concatenate([a2_src[0][:, None], pad,
                          a2_dst[0][:, None], pad], axis=1)[_P2]

    h1, oa1 = _tc_layer1(x, W1f, A1)
    num1, den1 = _sc_edge_pass(_NHEAD, _NHEAD * _NHID, 80, False)(
        adj, h1, oa1[:, :_AH], oa1[:, _AH:])
    h2, oa2 = _tc_mid(num1, den1, b1r, EXPAND, W2f, A2)
    num2, den2 = _sc_edge_pass(1, _NCLASS, 128, True)(
        adj, h2, oa2[:, :_AH], oa2[:, _AH:])
    return _tc_post(num2, den2)


# speedup vs baseline: 114.5994x; 1.0019x over previous
import jax
import jax.numpy as jnp
import numpy as np
from jax import lax
from jax.experimental import pallas as pl
from jax.experimental.pallas import tpu as pltpu
from jax.experimental.pallas import tpu_sc as plsc

_N = 10000
_E = 320000
_NFEAT = 128
_NHID = 16
_NHEAD = 8
_NCLASS = 64

_NC = 2
_NS = 16
_NW = _NC * _NS
_L = 16
_NP = 10240
_AH = 8
_BN = 1024


def _interleave_perm(width):
    return np.arange(width).reshape(-1, 2, 16).transpose(0, 2, 1).reshape(-1)


_P1 = _interleave_perm(_NFEAT)
_P2 = _interleave_perm(_NCLASS)


def _sc_edge_pass(H, HO, CH, use_bf16):
    n_chunks = _E // CH
    O = HO // H
    nz = _NP // CH // _NS

    mesh = plsc.VectorSubcoreMesh(core_axis_name="c", subcore_axis_name="s",
                                  num_cores=_NC, num_subcores=_NS)

    def body(adj, hmat, asrc, adst, num_out, den_out,
             ij0, ij1, ij2, hbf0, hbf1, hbf2, hrows0, hrows1, hrows2,
             asr0, asr1, asr2, adr0, adr1, adr2, wem0, wem1, wem2,
             num_s, den_s, gsem0, gsem1, gsem2, ssem0, ssem1, ssem2):
        c = lax.axis_index("c")
        s = lax.axis_index("s")
        wid = s * _NC + c
        zero16 = jnp.zeros((_L,), jnp.float32)
        iota = lax.iota(jnp.int32, _L)

        def zrow(k, carry):
            for j in range(HO // _L):
                hrows0[k, pl.ds(j * _L, _L)] = zero16
            return carry
        lax.fori_loop(0, CH, zrow, 0)
        r_pat = lax.shift_right_logical(iota, 3)
        c_pat = lax.bitwise_and(iota, 7)
        for i in range(CH // 2):
            plsc.store_scatter(wem0, [r_pat + 2 * i, c_pat], zero16)
            plsc.store_scatter(wem1, [r_pat + 2 * i, c_pat], zero16)
            plsc.store_scatter(wem2, [r_pat + 2 * i, c_pat], zero16)
        for j in range(nz):
            r = (s * nz + j) * CH
            pltpu.sync_copy(hrows0, num_s.at[pl.ds(r, CH)])
            pltpu.sync_copy(wem0, den_s.at[pl.ds(r, CH)])

        cols = [jnp.full((_L,), h, jnp.int32) for h in range(H)]
        hpat = lax.bitwise_and(iota, H - 1)
        bufs = ((ij0, hbf0, hrows0, asr0, adr0, wem0, gsem0, ssem0),
                (ij1, hbf1, hrows1, asr1, adr1, wem1, gsem1, ssem1),
                (ij2, hbf2, hrows2, asr2, adr2, wem2, gsem2, ssem2))

        def issue(i, b):
            ij, hbf, hrows, asr, adr, wem, gsem, ssem = bufs[b]

            @pl.when(i >= 3)
            def _():
                pltpu.make_async_copy(hrows, num_s.at[ij.at[1]], ssem).wait()
                pltpu.make_async_copy(wem, den_s.at[ij.at[1]], ssem).wait()

            base = (wid + _NW * i) * CH
            pltpu.sync_copy(adj.at[:, pl.ds(base, CH)], ij)
            pltpu.async_copy(hmat.at[ij.at[0]], hbf if use_bf16 else hrows, gsem)
            pltpu.async_copy(asrc.at[ij.at[0]], asr, gsem)
            pltpu.async_copy(adst.at[ij.at[1]], adr, gsem)

        def process(b):
            ij, hbf, hrows, asr, adr, wem, gsem, ssem = bufs[b]
            pltpu.make_async_copy(hmat.at[ij.at[0]],
                                  hbf if use_bf16 else hrows, gsem).wait()
            pltpu.make_async_copy(asrc.at[ij.at[0]], asr, gsem).wait()
            pltpu.make_async_copy(adst.at[ij.at[1]], adr, gsem).wait()
            for g in range(CH // _L):
                ridx = iota + (g * _L)
                for h in range(H):
                    ev = (plsc.load_gather(asr, [ridx, cols[h]])
                          + plsc.load_gather(adr, [ridx, cols[h]]))
                    ev = jnp.maximum(ev, 0.2 * ev)
                    plsc.store_scatter(wem, [ridx, cols[h]], jnp.exp(ev))

            @plsc.parallel_loop(0, CH, unroll=2)
            def mul_body(k):
                kvec = jnp.zeros((_L,), jnp.int32) + k
                wexp = plsc.load_gather(wem, [kvec, hpat])
                if use_bf16:
                    for j in range(HO // (2 * _L)):
                        hb = hbf[k, pl.ds(2 * _L * j, 2 * _L)]
                        va, vb = plsc.unpack(hb,
                                             format=plsc.PackFormat.INTERLEAVED)
                        hrows[k, pl.ds(2 * _L * j, _L)] = va * wexp
                        hrows[k, pl.ds(2 * _L * j + _L, _L)] = vb * wexp
                else:
                    for j in range(HO // _L):
                        sl = pl.ds(j * _L, _L)
                        hrows[k, sl] = hrows[k, sl] * wexp

            pltpu.async_copy(hrows, num_s.at[ij.at[1]], ssem, add=True)
            pltpu.async_copy(wem, den_s.at[ij.at[1]], ssem, add=True)

        n_my = (n_chunks - wid + _NW - 1) // _NW
        n_tri = (n_my + 2) // 3

        issue(0, 0)
        issue(1, 1)
        plsc.subcore_barrier()

        def triple(j, carry):
            i0 = 3 * j
            for t in range(3):
                it = i0 + t

                @pl.when(it < n_my)
                def _(t=t):
                    process(t)

                @pl.when(it + 2 < n_my)
                def _(t=t, it=it):
                    issue(it + 2, (t + 2) % 3)
            return carry

        lax.fori_loop(0, n_tri, triple, 0)

        for b in range(3):
            @pl.when(n_my > b)
            def _(b=b):
                ij, _hbf, hrows, _asr, _adr, wem, _g, ssem = bufs[b]
                pltpu.make_async_copy(hrows, num_s.at[ij.at[1]], ssem).wait()
                pltpu.make_async_copy(wem, den_s.at[ij.at[1]], ssem).wait()

        plsc.subcore_barrier()
        for j in range(nz):
            r = (s * nz + j) * CH
            pltpu.sync_copy(num_s.at[pl.ds(r, CH)], hrows0)
            pltpu.sync_copy(hrows0, num_out.at[c, pl.ds(r, CH)])
            pltpu.sync_copy(den_s.at[pl.ds(r, CH)], wem0)
            pltpu.sync_copy(wem0, den_out.at[c, pl.ds(r, CH)])

    return pl.kernel(
        body,
        out_type=(jax.ShapeDtypeStruct((_NC, _NP, HO), jnp.float32),
                  jax.ShapeDtypeStruct((_NC, _NP, _AH), jnp.float32)),
        mesh=mesh,
        compiler_params=pltpu.CompilerParams(needs_layout_passes=False,
                                             use_tc_tiling_on_sc=False),
        scratch_types=(
            [pltpu.VMEM((2, CH), jnp.int32)] * 3
            + [pltpu.VMEM((CH, HO) if use_bf16 else (2, _L),
                          jnp.bfloat16)] * 3
            + [pltpu.VMEM((CH, HO), jnp.float32)] * 3
            + [pltpu.VMEM((CH, _AH), jnp.float32)] * 6
            + [pltpu.VMEM((CH, _AH), jnp.float32)] * 3
            + [pltpu.VMEM_SHARED((_NP, HO), jnp.float32),
               pltpu.VMEM_SHARED((_NP, _AH), jnp.float32)]
            + [pltpu.SemaphoreType.DMA] * 6
        ),
    )


def _tc_layer1(x, W1f, A1):
    grid = ((_N + _BN - 1) // _BN,)

    def body(x_ref, w_ref, a_ref, h_ref, oa_ref):
        h = jnp.dot(x_ref[...], w_ref[...], preferred_element_type=jnp.float32)
        h_ref[...] = h
        oa_ref[...] = jnp.dot(h, a_ref[...], preferred_element_type=jnp.float32)

    return pl.pallas_call(
        body,
        grid=grid,
        in_specs=[pl.BlockSpec((_BN, _NFEAT), lambda i: (i, 0)),
                  pl.BlockSpec((_NFEAT, _NFEAT), lambda i: (0, 0)),
                  pl.BlockSpec((_NFEAT, 2 * _AH), lambda i: (0, 0))],
        out_specs=[pl.BlockSpec((_BN, _NFEAT), lambda i: (i, 0)),
                   pl.BlockSpec((_BN, 2 * _AH), lambda i: (i, 0))],
        out_shape=[jax.ShapeDtypeStruct((_N, _NFEAT), jnp.float32),
                   jax.ShapeDtypeStruct((_N, 2 * _AH), jnp.float32)],
    )(x, W1f, A1)


def _tc_mid(num1, den1, b1r, EXPAND, W2f, A2):
    grid = (_NP // _BN,)

    def body(n_ref, d_ref, b_ref, e_ref, w_ref, a_ref, h2_ref, oa_ref):
        num = n_ref[0] + n_ref[1]
        den = d_ref[0] + d_ref[1]
        den_w = jnp.dot(den, e_ref[...], preferred_element_type=jnp.float32)
        out1 = num / (den_w + 1e-16) + b_ref[...]
        h1 = jnp.where(out1 > 0, out1, jnp.exp(jnp.minimum(out1, 0.0)) - 1.0)
        h2 = jnp.dot(h1, w_ref[...], preferred_element_type=jnp.float32)
        h2_ref[...] = h2.astype(jnp.bfloat16)
        oa_ref[...] = jnp.dot(h2, a_ref[...], preferred_element_type=jnp.float32)

    return pl.pallas_call(
        body,
        grid=grid,
        in_specs=[pl.BlockSpec((_NC, _BN, _NFEAT), lambda i: (0, i, 0)),
                  pl.BlockSpec((_NC, _BN, _AH), lambda i: (0, i, 0)),
                  pl.BlockSpec((1, _NFEAT), lambda i: (0, 0)),
                  pl.BlockSpec((_AH, _NFEAT), lambda i: (0, 0)),
                  pl.BlockSpec((_NFEAT, _NCLASS), lambda i: (0, 0)),
                  pl.BlockSpec((_NCLASS, 2 * _AH), lambda i: (0, 0))],
        out_specs=[pl.BlockSpec((_BN, _NCLASS), lambda i: (i, 0)),
                   pl.BlockSpec((_BN, 2 * _AH), lambda i: (i, 0))],
        out_shape=[jax.ShapeDtypeStruct((_N, _NCLASS), jnp.bfloat16),
                   jax.ShapeDtypeStruct((_N, 2 * _AH), jnp.float32)],
    )(num1, den1, b1r, EXPAND, W2f, A2)


def _tc_post(num2, den2):
    grid = (_NP // _BN,)

    def body(n_ref, d_ref, o_ref):
        num = n_ref[0] + n_ref[1]
        den = d_ref[0, :, 0:1] + d_ref[1, :, 0:1]
        o_ref[...] = num / (den + 1e-16)

    return pl.pallas_call(
        body,
        grid=grid,
        in_specs=[pl.BlockSpec((_NC, _BN, _NCLASS), lambda i: (0, i, 0)),
                  pl.BlockSpec((_NC, _BN, _AH), lambda i: (0, i, 0))],
        out_specs=pl.BlockSpec((_BN, _NCLASS), lambda i: (i, 0)),
        out_shape=jax.ShapeDtypeStruct((_N, _NCLASS), jnp.float32),
    )(num2, den2)


def kernel(x, adj, W1, a1_src, a1_dst, b1, W2, a2_src, a2_dst):
    W1f = W1.transpose(1, 2, 0).reshape(_NFEAT, _NHEAD * _NHID)
    eye = jnp.eye(_NHEAD, dtype=jnp.float32)
    EXPAND = jnp.tile(eye, (1, _NHID))
    sel = EXPAND.T
    A1 = jnp.concatenate([sel * a1_src.T.reshape(-1)[:, None],
                          sel * a1_dst.T.reshape(-1)[:, None]], axis=1)
    b1r = b1.reshape(_NHEAD, _NHID).T.reshape(1, _NFEAT)
    W2f = W2[0].reshape(_NHEAD, _NHID, _NCLASS).transpose(1, 0, 2)
    W2f = W2f.reshape(_NFEAT, _NCLASS)[:, _P2]
    pad = jnp.zeros((_NCLASS, _AH - 1), jnp.float32)
    A2 = jnp.concatenate([a2_src[0][:, None], pad,
                          a2_dst[0][:, None], pad], axis=1)[_P2]

    h1, oa1 = _tc_layer1(x, W1f, A1)
    num1, den1 = _sc_edge_pass(_NHEAD, _NHEAD * _NHID, 80, False)(
        adj, h1, oa1[:, :_AH], oa1[:, _AH:])
    h2, oa2 = _tc_mid(num1, den1, b1r, EXPAND, W2f, A2)
    num2, den2 = _sc_edge_pass(1, _NCLASS, 128, True)(
        adj, h2, oa2[:, :_AH], oa2[:, _AH:])
    return _tc_post(num2, den2)
